# baseline (speedup 1.0000x reference)
import functools

import jax
import jax.numpy as jnp
from jax import lax
from jax.experimental import pallas as pl
from jax.experimental.pallas import tpu as pltpu
from jax.experimental.pallas import tpu_sc as plsc

_NUM_FIELDS = 26
_VOCAB1 = 100001
_DIM = 32
_BATCH = 4096

_NC = 2
_NS = 16
_DB = _DIM // 8
_SLABS = _NUM_FIELDS * _DB
_SLABS_PER_SC = _SLABS // _NC
_ROUNDS = _SLABS_PER_SC // 2
_GVEC = _BATCH // 16
_VMAIN = (_VOCAB1 // 128) * 128
_VTAIL = 128
_VCH = 4096
_NFULL = _VMAIN // _VCH
_VREM = _VMAIN - _NFULL * _VCH
_VSUB = _VCH // 8


def _lookup_body(idx_hbm, table_hbm, tail_hbm, out_hbm,
                 idx_sp, slab_sp, oslab_sp, idx_v, row_v, out_v, dsem):
    c = lax.axis_index("c")
    s = lax.axis_index("s")
    grp = s // 8
    sg = s % 8

    @pl.when(s == 0)
    def _():
        pltpu.sync_copy(idx_hbm, idx_sp)
    plsc.subcore_barrier()

    def round_body(r, carry):
        slab = c * _SLABS_PER_SC + 2 * r + grp
        f = slab // _DB
        d0 = pl.multiple_of((slab % _DB) * 8, 8)

        pltpu.sync_copy(idx_sp.at[f], idx_v)

        def start_chunk(k, buf):
            off = k * _VCH
            return pltpu.async_copy(
                table_hbm.at[f, pl.ds(d0, 8), pl.ds(off + sg * _VSUB, _VSUB)],
                slab_sp.at[grp, buf, :, pl.ds(sg * _VSUB, _VSUB)],
                dsem,
            )

        pending = start_chunk(0, 0)
        for k in range(_NFULL):
            pending.wait()
            plsc.subcore_barrier()
            if k + 1 < _NFULL:
                pending = start_chunk(k + 1, (k + 1) % 2)
            else:
                @pl.when(sg == 0)
                def _():
                    pltpu.sync_copy(
                        table_hbm.at[f, pl.ds(d0, 8),
                                     pl.ds(_NFULL * _VCH, _VREM)],
                        slab_sp.at[grp, (k + 1) % 2, :, pl.ds(0, _VREM)],
                    )
                @pl.when(sg == 1)
                def _():
                    pltpu.sync_copy(
                        tail_hbm.at[f, pl.ds(d0, 8)],
                        slab_sp.at[grp, (k + 1) % 2, :,
                                   pl.ds(_VREM, _VTAIL)],
                    )
            pltpu.sync_copy(slab_sp.at[grp, k % 2, sg],
                            row_v.at[pl.ds(k * _VCH, _VCH)])
        plsc.subcore_barrier()
        pltpu.sync_copy(
            slab_sp.at[grp, _NFULL % 2, sg, pl.ds(0, _VREM + _VTAIL)],
            row_v.at[pl.ds(_NFULL * _VCH, _VREM + _VTAIL)])

        def g(i, carry2):
            for u in range(4):
                sl = pl.ds(i * 64 + u * 16, 16)
                out_v[sl] = plsc.load_gather(row_v, [idx_v[sl]])
            return carry2

        lax.fori_loop(0, _GVEC // 4, g, 0)

        pltpu.sync_copy(out_v, oslab_sp.at[grp, sg])
        plsc.subcore_barrier()

        @pl.when(sg == 0)
        def _():
            pltpu.sync_copy(oslab_sp.at[grp], out_hbm.at[f, pl.ds(d0, 8)])
        plsc.subcore_barrier()
        return carry

    lax.fori_loop(0, _ROUNDS, round_body, 0)


_mesh = plsc.VectorSubcoreMesh(core_axis_name="c", subcore_axis_name="s")

_lookup = functools.partial(
    pl.kernel,
    out_type=jax.ShapeDtypeStruct((_NUM_FIELDS, _DIM, _BATCH), jnp.float32),
    mesh=_mesh,
    scratch_types=[
        pltpu.VMEM_SHARED((32, _BATCH), jnp.int32),
        pltpu.VMEM_SHARED((2, 2, 8, _VCH), jnp.float32),
        pltpu.VMEM_SHARED((2, 8, _BATCH), jnp.float32),
        pltpu.VMEM((_BATCH,), jnp.int32),
        pltpu.VMEM((_VMAIN + _VTAIL,), jnp.float32),
        pltpu.VMEM((_BATCH,), jnp.float32),
        pltpu.SemaphoreType.DMA,
    ],
    compiler_params=pltpu.CompilerParams(needs_layout_passes=False),
)(_lookup_body)


@jax.jit
def kernel(sparse_inputs, tables):
    idx_t = jnp.pad(sparse_inputs.T, ((0, 32 - _NUM_FIELDS), (0, 0)))
    tab_t = jnp.transpose(tables, (0, 2, 1))
    tail = jnp.pad(tab_t[:, :, _VMAIN:], ((0, 0), (0, 0), (0, _VTAIL - (_VOCAB1 - _VMAIN))))
    out = _lookup(idx_t, tab_t, tail)
    return tuple(out[i].T for i in range(_NUM_FIELDS))


# speedup vs baseline: 1.3852x; 1.3852x over previous
import functools

import jax
import jax.numpy as jnp
from jax import lax
from jax.experimental import pallas as pl
from jax.experimental.pallas import tpu as pltpu
from jax.experimental.pallas import tpu_sc as plsc

_NUM_FIELDS = 26
_VOCAB1 = 100001
_DIM = 32
_BATCH = 4096

_NC = 2
_NS = 16
_DB = _DIM // 8
_SLABS = _NUM_FIELDS * _DB
_SLABS_PER_SC = _SLABS // _NC
_ROUNDS = _SLABS_PER_SC // 2
_GVEC = _BATCH // 16
_VMAIN = (_VOCAB1 // 128) * 128
_VTAIL = 128
_VCH = 8192
_NFULL = _VMAIN // _VCH
_VREM = _VMAIN - _NFULL * _VCH
_VSUB = _VCH // 8


def _lookup_body(idx_hbm, table_hbm, tail_hbm, out_hbm,
                 slab_sp, oslab_sp, idx_v, row_v, out_v, dsem):
    c = lax.axis_index("c")
    s = lax.axis_index("s")
    grp = s // 8
    sg = s % 8

    def round_body(r, carry):
        slab = c * _SLABS_PER_SC + 2 * r + grp
        f = slab // _DB
        d0 = pl.multiple_of((slab % _DB) * 8, 8)

        pltpu.sync_copy(idx_hbm.at[f], idx_v)

        def start_chunk(k, buf):
            off = k * _VCH
            return pltpu.async_copy(
                table_hbm.at[f, pl.ds(d0, 8), pl.ds(off + sg * _VSUB, _VSUB)],
                slab_sp.at[grp, buf, :, pl.ds(sg * _VSUB, _VSUB)],
                dsem,
            )

        pending = start_chunk(0, 0)
        for k in range(_NFULL):
            pending.wait()
            plsc.subcore_barrier()
            if k + 1 < _NFULL:
                pending = start_chunk(k + 1, (k + 1) % 2)
            else:
                @pl.when(sg == 0)
                def _():
                    pltpu.sync_copy(
                        table_hbm.at[f, pl.ds(d0, 8),
                                     pl.ds(_NFULL * _VCH, _VREM)],
                        slab_sp.at[grp, (k + 1) % 2, :, pl.ds(0, _VREM)],
                    )
                @pl.when(sg == 1)
                def _():
                    pltpu.sync_copy(
                        tail_hbm.at[f, pl.ds(d0, 8)],
                        slab_sp.at[grp, (k + 1) % 2, :,
                                   pl.ds(_VREM, _VTAIL)],
                    )
            pltpu.sync_copy(slab_sp.at[grp, k % 2, sg],
                            row_v.at[pl.ds(k * _VCH, _VCH)])
        plsc.subcore_barrier()
        pltpu.sync_copy(
            slab_sp.at[grp, _NFULL % 2, sg, pl.ds(0, _VREM + _VTAIL)],
            row_v.at[pl.ds(_NFULL * _VCH, _VREM + _VTAIL)])

        def g(rr, carry2):
            for u in range(8):
                out_v[pl.ds(rr * 128 + u * 16, 16)] = plsc.load_gather(
                    row_v, [idx_v[rr, pl.ds(u * 16, 16)]])
            return carry2

        lax.fori_loop(0, 32, g, 0)

        pltpu.sync_copy(out_v, oslab_sp.at[grp, sg])
        plsc.subcore_barrier()

        @pl.when(sg == 0)
        def _():
            pltpu.sync_copy(oslab_sp.at[grp], out_hbm.at[f, pl.ds(d0, 8)])
        plsc.subcore_barrier()
        return carry

    lax.fori_loop(0, _ROUNDS, round_body, 0)


_mesh = plsc.VectorSubcoreMesh(core_axis_name="c", subcore_axis_name="s")

_lookup = functools.partial(
    pl.kernel,
    out_type=jax.ShapeDtypeStruct((_NUM_FIELDS, _DIM, _BATCH), jnp.float32),
    mesh=_mesh,
    scratch_types=[
        pltpu.VMEM_SHARED((2, 2, 8, _VCH), jnp.float32),
        pltpu.VMEM_SHARED((2, 8, _BATCH), jnp.float32),
        pltpu.VMEM((32, 128), jnp.int32),
        pltpu.VMEM((_VMAIN + _VTAIL,), jnp.float32),
        pltpu.VMEM((_BATCH,), jnp.float32),
        pltpu.SemaphoreType.DMA,
    ],
    compiler_params=pltpu.CompilerParams(needs_layout_passes=False),
)(_lookup_body)


@jax.jit
def kernel(sparse_inputs, tables):
    idx_t = sparse_inputs.T.reshape(_NUM_FIELDS, 32, 128)
    tab_t = jnp.transpose(tables, (0, 2, 1))
    tail = jnp.pad(tab_t[:, :, _VMAIN:], ((0, 0), (0, 0), (0, _VTAIL - (_VOCAB1 - _VMAIN))))
    out = _lookup(idx_t, tab_t, tail)
    return tuple(out[i].T for i in range(_NUM_FIELDS))


# speedup vs baseline: 1.4713x; 1.0621x over previous
import functools

import jax
import jax.numpy as jnp
from jax import lax
from jax.experimental import pallas as pl
from jax.experimental.pallas import tpu as pltpu
from jax.experimental.pallas import tpu_sc as plsc

_NUM_FIELDS = 26
_VOCAB1 = 100001
_DIM = 32
_BATCH = 4096

_NC = 2
_NS = 16
_DB = _DIM // 8
_SLABS = _NUM_FIELDS * _DB
_SLABS_PER_SC = _SLABS // _NC
_ROUNDS = _SLABS_PER_SC // 2
_GVEC = _BATCH // 16
_VMAIN = (_VOCAB1 // 128) * 128
_VTAIL = 128
_VCH = 9216
_NFULL = _VMAIN // _VCH
_VREM = _VMAIN - _NFULL * _VCH
_VSUB = _VCH // 8


def _lookup_body(idx_hbm, table_hbm, tail_hbm, out_hbm,
                 slab_sp, oslab_sp, idx_v, row_v, out_v, dsem):
    c = lax.axis_index("c")
    s = lax.axis_index("s")
    grp = s // 8
    sg = s % 8

    def round_body(r, carry):
        slab = c * _SLABS_PER_SC + 2 * r + grp
        f = slab // _DB
        d0 = pl.multiple_of((slab % _DB) * 8, 8)

        pltpu.sync_copy(idx_hbm.at[f], idx_v)

        def start_chunk(k, buf):
            off = k * _VCH
            return pltpu.async_copy(
                table_hbm.at[f, pl.ds(d0, 8), pl.ds(off + sg * _VSUB, _VSUB)],
                slab_sp.at[grp, buf, :, pl.ds(sg * _VSUB, _VSUB)],
                dsem,
            )

        pending = start_chunk(0, 0)
        for k in range(_NFULL):
            pending.wait()
            plsc.subcore_barrier()
            if k + 1 < _NFULL:
                pending = start_chunk(k + 1, (k + 1) % 2)
            else:
                @pl.when(sg == 0)
                def _():
                    pltpu.sync_copy(
                        table_hbm.at[f, pl.ds(d0, 8),
                                     pl.ds(_NFULL * _VCH, _VREM)],
                        slab_sp.at[grp, (k + 1) % 2, :, pl.ds(0, _VREM)],
                    )
                @pl.when(sg == 1)
                def _():
                    pltpu.sync_copy(
                        tail_hbm.at[f, pl.ds(d0, 8)],
                        slab_sp.at[grp, (k + 1) % 2, :,
                                   pl.ds(_VREM, _VTAIL)],
                    )
            pltpu.sync_copy(slab_sp.at[grp, k % 2, sg],
                            row_v.at[pl.ds(k * _VCH, _VCH)])
        plsc.subcore_barrier()
        pltpu.sync_copy(
            slab_sp.at[grp, _NFULL % 2, sg, pl.ds(0, _VREM + _VTAIL)],
            row_v.at[pl.ds(_NFULL * _VCH, _VREM + _VTAIL)])

        def g(rr, carry2):
            for u in range(8):
                out_v[pl.ds(rr * 128 + u * 16, 16)] = plsc.load_gather(
                    row_v, [idx_v[rr, pl.ds(u * 16, 16)]])
            return carry2

        lax.fori_loop(0, 32, g, 0)

        pltpu.sync_copy(out_v, oslab_sp.at[grp, sg])
        plsc.subcore_barrier()

        @pl.when(sg == 0)
        def _():
            pltpu.sync_copy(oslab_sp.at[grp], out_hbm.at[f, pl.ds(d0, 8)])
        plsc.subcore_barrier()
        return carry

    lax.fori_loop(0, _ROUNDS, round_body, 0)


_mesh = plsc.VectorSubcoreMesh(core_axis_name="c", subcore_axis_name="s")

_lookup = functools.partial(
    pl.kernel,
    out_type=jax.ShapeDtypeStruct((_NUM_FIELDS, _DIM, _BATCH), jnp.float32),
    mesh=_mesh,
    scratch_types=[
        pltpu.VMEM_SHARED((2, 2, 8, _VCH), jnp.float32),
        pltpu.VMEM_SHARED((2, 8, _BATCH), jnp.float32),
        pltpu.VMEM((32, 128), jnp.int32),
        pltpu.VMEM((_VMAIN + _VTAIL,), jnp.float32),
        pltpu.VMEM((_BATCH,), jnp.float32),
        pltpu.SemaphoreType.DMA,
    ],
    compiler_params=pltpu.CompilerParams(needs_layout_passes=False),
)(_lookup_body)


@jax.jit
def kernel(sparse_inputs, tables):
    idx_t = sparse_inputs.T.reshape(_NUM_FIELDS, 32, 128)
    tab_t = jnp.transpose(tables, (0, 2, 1))
    tail = jnp.pad(tab_t[:, :, _VMAIN:], ((0, 0), (0, 0), (0, _VTAIL - (_VOCAB1 - _VMAIN))))
    out = _lookup(idx_t, tab_t, tail)
    return tuple(out[i].T for i in range(_NUM_FIELDS))


# speedup vs baseline: 1.8107x; 1.2307x over previous
import functools

import jax
import jax.numpy as jnp
from jax import lax
from jax.experimental import pallas as pl
from jax.experimental.pallas import tpu as pltpu
from jax.experimental.pallas import tpu_sc as plsc

_NUM_FIELDS = 26
_VOCAB1 = 100001
_DIM = 32
_BATCH = 4096

_NC = 2
_NS = 16
_DB = _DIM // 8
_SLABS = _NUM_FIELDS * _DB
_SLABS_PER_SC = _SLABS // _NC
_ROUNDS = _SLABS_PER_SC // 2
_GVEC = _BATCH // 16
_VMAIN = (_VOCAB1 // 128) * 128
_VTAIL = 128
_VCH = 6144
_NBUF = 3
_NFULL = _VMAIN // _VCH
_VREM = _VMAIN - _NFULL * _VCH
_VSUB = _VCH // 8


def _lookup_body(idx_hbm, table_hbm, tail_hbm, out_hbm,
                 slab_sp, oslab_sp, idx_v, row_v, out_v, dsem):
    c = lax.axis_index("c")
    s = lax.axis_index("s")
    grp = s // 8
    sg = s % 8

    def round_body(r, carry):
        slab = c * _SLABS_PER_SC + 2 * r + grp
        f = slab // _DB
        d0 = pl.multiple_of((slab % _DB) * 8, 8)

        pltpu.sync_copy(idx_hbm.at[f], idx_v)

        def start_chunk(k, buf):
            off = k * _VCH
            return pltpu.async_copy(
                table_hbm.at[f, pl.ds(d0, 8), pl.ds(off + sg * _VSUB, _VSUB)],
                slab_sp.at[grp, buf, :, pl.ds(sg * _VSUB, _VSUB)],
                dsem,
            )

        pend = [start_chunk(0, 0), None]
        pend[1] = start_chunk(1, 1)
        for k in range(_NFULL):
            pend[k % 2].wait()
            plsc.subcore_barrier()
            if k + 2 < _NFULL:
                pend[k % 2] = start_chunk(k + 2, (k + 2) % _NBUF)
            elif k + 2 == _NFULL:
                rbuf = (k + 2) % _NBUF
                @pl.when(sg == 0)
                def _():
                    pltpu.sync_copy(
                        table_hbm.at[f, pl.ds(d0, 8),
                                     pl.ds(_NFULL * _VCH, _VREM)],
                        slab_sp.at[grp, rbuf, :, pl.ds(0, _VREM)],
                    )
                @pl.when(sg == 1)
                def _():
                    pltpu.sync_copy(
                        tail_hbm.at[f, pl.ds(d0, 8)],
                        slab_sp.at[grp, rbuf, :, pl.ds(_VREM, _VTAIL)],
                    )
            pltpu.sync_copy(slab_sp.at[grp, k % _NBUF, sg],
                            row_v.at[pl.ds(k * _VCH, _VCH)])
        plsc.subcore_barrier()
        rbuf = _NFULL % _NBUF
        pltpu.sync_copy(
            slab_sp.at[grp, rbuf, sg, pl.ds(0, _VREM + _VTAIL)],
            row_v.at[pl.ds(_NFULL * _VCH, _VREM + _VTAIL)])

        def g(rr, carry2):
            for u in range(8):
                out_v[pl.ds(rr * 128 + u * 16, 16)] = plsc.load_gather(
                    row_v, [idx_v[rr, pl.ds(u * 16, 16)]])
            return carry2

        lax.fori_loop(0, 32, g, 0)

        pltpu.sync_copy(out_v, oslab_sp.at[grp, sg])
        plsc.subcore_barrier()

        @pl.when(sg == 0)
        def _():
            pltpu.sync_copy(oslab_sp.at[grp], out_hbm.at[f, pl.ds(d0, 8)])
        plsc.subcore_barrier()
        return carry

    lax.fori_loop(0, _ROUNDS, round_body, 0)


_mesh = plsc.VectorSubcoreMesh(core_axis_name="c", subcore_axis_name="s")

_lookup = functools.partial(
    pl.kernel,
    out_type=jax.ShapeDtypeStruct((_NUM_FIELDS, _DIM, _BATCH), jnp.float32),
    mesh=_mesh,
    scratch_types=[
        pltpu.VMEM_SHARED((2, _NBUF, 8, _VCH), jnp.float32),
        pltpu.VMEM_SHARED((2, 8, _BATCH), jnp.float32),
        pltpu.VMEM((32, 128), jnp.int32),
        pltpu.VMEM((_VMAIN + _VTAIL,), jnp.float32),
        pltpu.VMEM((_BATCH,), jnp.float32),
        pltpu.SemaphoreType.DMA,
    ],
    compiler_params=pltpu.CompilerParams(needs_layout_passes=False),
)(_lookup_body)


@jax.jit
def kernel(sparse_inputs, tables):
    idx_t = sparse_inputs.T.reshape(_NUM_FIELDS, 32, 128)
    tab_t = jnp.transpose(tables, (0, 2, 1))
    tail = jnp.pad(tab_t[:, :, _VMAIN:], ((0, 0), (0, 0), (0, _VTAIL - (_VOCAB1 - _VMAIN))))
    out = _lookup(idx_t, tab_t, tail)
    return tuple(out[i].T for i in range(_NUM_FIELDS))


# speedup vs baseline: 1.9266x; 1.0640x over previous
import functools

import jax
import jax.numpy as jnp
from jax import lax
from jax.experimental import pallas as pl
from jax.experimental.pallas import tpu as pltpu
from jax.experimental.pallas import tpu_sc as plsc

_NUM_FIELDS = 26
_VOCAB1 = 100001
_DIM = 32
_BATCH = 4096

_NC = 2
_NS = 16
_DB = _DIM // 8
_SLABS = _NUM_FIELDS * _DB
_SLABS_PER_SC = _SLABS // _NC
_ROUNDS = _SLABS_PER_SC // 2
_GVEC = _BATCH // 16
_VMAIN = (_VOCAB1 // 128) * 128
_VTAIL = 128
_VCH = 6144
_NBUF = 3
_NFULL = _VMAIN // _VCH
_VREM = _VMAIN - _NFULL * _VCH
_VSUB = _VCH // 8


def _lookup_body(idx_hbm, table_hbm, tail_hbm, out_hbm,
                 slab_sp, oslab_sp, idx_v, row_v, out_v, dsem):
    c = lax.axis_index("c")
    s = lax.axis_index("s")
    grp = s // 8
    sg = s % 8

    def chunk_copy(f, d0, k, buf):
        off = k * _VCH
        return pltpu.make_async_copy(
            table_hbm.at[f, pl.ds(d0, 8), pl.ds(off + sg * _VSUB, _VSUB)],
            slab_sp.at[grp, buf, :, pl.ds(sg * _VSUB, _VSUB)],
            dsem,
        )

    def slab_coords(slab):
        return slab // _DB, pl.multiple_of((slab % _DB) * 8, 8)

    f0, d00 = slab_coords(c * _SLABS_PER_SC + grp)
    chunk_copy(f0, d00, 0, 0).start()
    chunk_copy(f0, d00, 1, 1).start()

    def round_body(r, carry):
        slab = c * _SLABS_PER_SC + 2 * r + grp
        f, d0 = slab_coords(slab)

        pltpu.sync_copy(idx_hbm.at[f], idx_v)

        for k in range(_NFULL):
            chunk_copy(f, d0, k, k % _NBUF).wait()
            plsc.subcore_barrier()
            if k + 2 < _NFULL:
                chunk_copy(f, d0, k + 2, (k + 2) % _NBUF).start()
            elif k + 2 == _NFULL:
                rbuf = (k + 2) % _NBUF
                @pl.when(sg == 0)
                def _():
                    pltpu.sync_copy(
                        table_hbm.at[f, pl.ds(d0, 8),
                                     pl.ds(_NFULL * _VCH, _VREM)],
                        slab_sp.at[grp, rbuf, :, pl.ds(0, _VREM)],
                    )
                @pl.when(sg == 1)
                def _():
                    pltpu.sync_copy(
                        tail_hbm.at[f, pl.ds(d0, 8)],
                        slab_sp.at[grp, rbuf, :, pl.ds(_VREM, _VTAIL)],
                    )
            pltpu.sync_copy(slab_sp.at[grp, k % _NBUF, sg],
                            row_v.at[pl.ds(k * _VCH, _VCH)])
        plsc.subcore_barrier()
        rbuf = _NFULL % _NBUF
        pltpu.sync_copy(
            slab_sp.at[grp, rbuf, sg, pl.ds(0, _VREM + _VTAIL)],
            row_v.at[pl.ds(_NFULL * _VCH, _VREM + _VTAIL)])
        plsc.subcore_barrier()

        @pl.when(r + 1 < _ROUNDS)
        def _():
            nf, nd0 = slab_coords(slab + 2)
            chunk_copy(nf, nd0, 0, 0).start()
            chunk_copy(nf, nd0, 1, 1).start()

        def g(rr, carry2):
            for u in range(8):
                out_v[pl.ds(rr * 128 + u * 16, 16)] = plsc.load_gather(
                    row_v, [idx_v[rr, pl.ds(u * 16, 16)]])
            return carry2

        lax.fori_loop(0, 32, g, 0)

        pltpu.sync_copy(out_v, oslab_sp.at[grp, sg])
        plsc.subcore_barrier()

        @pl.when(sg == 0)
        def _():
            pltpu.sync_copy(oslab_sp.at[grp], out_hbm.at[f, pl.ds(d0, 8)])
        plsc.subcore_barrier()
        return carry

    lax.fori_loop(0, _ROUNDS, round_body, 0)


_mesh = plsc.VectorSubcoreMesh(core_axis_name="c", subcore_axis_name="s")

_lookup = functools.partial(
    pl.kernel,
    out_type=jax.ShapeDtypeStruct((_NUM_FIELDS, _DIM, _BATCH), jnp.float32),
    mesh=_mesh,
    scratch_types=[
        pltpu.VMEM_SHARED((2, _NBUF, 8, _VCH), jnp.float32),
        pltpu.VMEM_SHARED((2, 8, _BATCH), jnp.float32),
        pltpu.VMEM((32, 128), jnp.int32),
        pltpu.VMEM((_VMAIN + _VTAIL,), jnp.float32),
        pltpu.VMEM((_BATCH,), jnp.float32),
        pltpu.SemaphoreType.DMA,
    ],
    compiler_params=pltpu.CompilerParams(needs_layout_passes=False),
)(_lookup_body)


@jax.jit
def kernel(sparse_inputs, tables):
    idx_t = sparse_inputs.T.reshape(_NUM_FIELDS, 32, 128)
    tab_t = jnp.transpose(tables, (0, 2, 1))
    tail = jnp.pad(tab_t[:, :, _VMAIN:], ((0, 0), (0, 0), (0, _VTAIL - (_VOCAB1 - _VMAIN))))
    out = _lookup(idx_t, tab_t, tail)
    return tuple(out[i].T for i in range(_NUM_FIELDS))


# speedup vs baseline: 2.1833x; 1.1332x over previous
import functools

import jax
import jax.numpy as jnp
from jax import lax
from jax.experimental import pallas as pl
from jax.experimental.pallas import tpu as pltpu
from jax.experimental.pallas import tpu_sc as plsc

_NUM_FIELDS = 26
_VOCAB1 = 100001
_DIM = 32
_BATCH = 4096

_NC = 2
_NS = 16
_DB = _DIM // 8
_SLABS = _NUM_FIELDS * _DB
_SLABS_PER_SC = _SLABS // _NC
_ROUNDS = _SLABS_PER_SC // 2
_GVEC = _BATCH // 16
_VMAIN = (_VOCAB1 // 128) * 128
_VTAIL = 128
_VCH = 6144
_NBUF = 3
_NFULL = _VMAIN // _VCH
_VREM = _VMAIN - _NFULL * _VCH
_VSUB = _VCH // 8


def _lookup_body(idx_hbm, table_hbm, tail_hbm, out_hbm,
                 slab_sp, oslab_sp, idx_v, row_v, out_v, dsem, isem):
    c = lax.axis_index("c")
    s = lax.axis_index("s")
    grp = s // 8
    sg = s % 8

    def chunk_copy(f, d0, k, buf):
        off = k * _VCH
        return pltpu.make_async_copy(
            table_hbm.at[f, pl.ds(d0, 8), pl.ds(off + sg * _VSUB, _VSUB)],
            slab_sp.at[grp, buf, :, pl.ds(sg * _VSUB, _VSUB)],
            dsem,
        )

    def slab_coords(slab):
        return slab // _DB, pl.multiple_of((slab % _DB) * 8, 8)

    def rem_copy(f, d0, rbuf):
        return pltpu.make_async_copy(
            table_hbm.at[f, pl.ds(d0, 8), pl.ds(_NFULL * _VCH, _VREM)],
            slab_sp.at[grp, rbuf, :, pl.ds(0, _VREM)],
            dsem,
        )

    def tail_copy(f, d0, rbuf):
        return pltpu.make_async_copy(
            tail_hbm.at[f, pl.ds(d0, 8)],
            slab_sp.at[grp, rbuf, :, pl.ds(_VREM, _VTAIL)],
            dsem,
        )

    f0, d00 = slab_coords(c * _SLABS_PER_SC + grp)
    chunk_copy(f0, d00, 0, 0).start()
    chunk_copy(f0, d00, 1, 1).start()

    def round_body(r, carry):
        slab = c * _SLABS_PER_SC + 2 * r + grp
        f, d0 = slab_coords(slab)

        idx_cp = pltpu.make_async_copy(idx_hbm.at[f], idx_v, isem)
        idx_cp.start()

        for k in range(_NFULL):
            chunk_copy(f, d0, k, k % _NBUF).wait()
            plsc.subcore_barrier()
            if k + 2 < _NFULL:
                chunk_copy(f, d0, k + 2, (k + 2) % _NBUF).start()
            elif k + 2 == _NFULL:
                rbuf = (k + 2) % _NBUF
                @pl.when(sg == 0)
                def _():
                    rem_copy(f, d0, rbuf).start()
                @pl.when(sg == 1)
                def _():
                    tail_copy(f, d0, rbuf).start()
            pltpu.sync_copy(slab_sp.at[grp, k % _NBUF, sg],
                            row_v.at[pl.ds(k * _VCH, _VCH)])
        rbuf = _NFULL % _NBUF
        @pl.when(sg == 0)
        def _():
            rem_copy(f, d0, rbuf).wait()
        @pl.when(sg == 1)
        def _():
            tail_copy(f, d0, rbuf).wait()
        plsc.subcore_barrier()
        pltpu.sync_copy(
            slab_sp.at[grp, rbuf, sg, pl.ds(0, _VREM + _VTAIL)],
            row_v.at[pl.ds(_NFULL * _VCH, _VREM + _VTAIL)])
        plsc.subcore_barrier()

        @pl.when(r + 1 < _ROUNDS)
        def _():
            nf, nd0 = slab_coords(slab + 2)
            chunk_copy(nf, nd0, 0, 0).start()
            chunk_copy(nf, nd0, 1, 1).start()

        idx_cp.wait()

        def g(rr, carry2):
            for u in range(8):
                out_v[pl.ds(rr * 128 + u * 16, 16)] = plsc.load_gather(
                    row_v, [idx_v[rr, pl.ds(u * 16, 16)]])
            return carry2

        lax.fori_loop(0, 32, g, 0)

        pltpu.sync_copy(out_v, oslab_sp.at[grp, sg])
        plsc.subcore_barrier()

        @pl.when(sg == 0)
        def _():
            pltpu.sync_copy(oslab_sp.at[grp], out_hbm.at[f, pl.ds(d0, 8)])
        plsc.subcore_barrier()
        return carry

    lax.fori_loop(0, _ROUNDS, round_body, 0)


_mesh = plsc.VectorSubcoreMesh(core_axis_name="c", subcore_axis_name="s")

_lookup = functools.partial(
    pl.kernel,
    out_type=jax.ShapeDtypeStruct((_NUM_FIELDS, _DIM, _BATCH), jnp.float32),
    mesh=_mesh,
    scratch_types=[
        pltpu.VMEM_SHARED((2, _NBUF, 8, _VCH), jnp.float32),
        pltpu.VMEM_SHARED((2, 8, _BATCH), jnp.float32),
        pltpu.VMEM((32, 128), jnp.int32),
        pltpu.VMEM((_VMAIN + _VTAIL,), jnp.float32),
        pltpu.VMEM((_BATCH,), jnp.float32),
        pltpu.SemaphoreType.DMA,
        pltpu.SemaphoreType.DMA,
    ],
    compiler_params=pltpu.CompilerParams(needs_layout_passes=False),
)(_lookup_body)


@jax.jit
def kernel(sparse_inputs, tables):
    idx_t = sparse_inputs.T.reshape(_NUM_FIELDS, 32, 128)
    tab_t = jnp.transpose(tables, (0, 2, 1))
    tail = jnp.pad(tab_t[:, :, _VMAIN:], ((0, 0), (0, 0), (0, _VTAIL - (_VOCAB1 - _VMAIN))))
    out = _lookup(idx_t, tab_t, tail)
    return tuple(out[i].T for i in range(_NUM_FIELDS))
